# baseline (speedup 1.0000x reference)
import functools

import jax
import jax.numpy as jnp
from jax import lax
from jax.experimental import pallas as pl
from jax.experimental.pallas import tpu as pltpu
from jax.experimental.pallas import tpu_sc as plsc

NUM_USERS = 25000
NUM_ITEMS = 25000
NUM_NODES = NUM_USERS + NUM_ITEMS
EMBED_DIM = 64
NUM_EDGES = 800000
NUM_LAYERS = 3

HALF = 25000
ACC_ROWS = 25088
ROWS_PER_TILE = ACC_ROWS // 16
CHUNK = 128
NCHUNK = 396
EDGES_PER_TILE = CHUNK * NCHUNK
E_PAD = EDGES_PER_TILE * 16
NSLOT = 3


def _propagate_layer(emb, src, dst, w, zeros_acc):
    mesh = plsc.VectorSubcoreMesh(core_axis_name="c", subcore_axis_name="s")

    @functools.partial(
        pl.kernel,
        mesh=mesh,
        compiler_params=pltpu.CompilerParams(needs_layout_passes=False,
                                             use_tc_tiling_on_sc=False),
        out_type=jax.ShapeDtypeStruct((NUM_NODES, EMBED_DIM), jnp.float32),
        scratch_types=[
            pltpu.VMEM_SHARED((ACC_ROWS, EMBED_DIM), jnp.float32),
        ] + [pltpu.VMEM((CHUNK,), jnp.int32)] * NSLOT
          + [pltpu.VMEM((CHUNK,), jnp.int32)] * NSLOT
          + [pltpu.VMEM((CHUNK,), jnp.float32)] * NSLOT
          + [pltpu.VMEM((CHUNK,), jnp.int32)] * NSLOT
          + [pltpu.VMEM((CHUNK, EMBED_DIM), jnp.float32)] * NSLOT
          + [pltpu.SemaphoreType.DMA] * (3 * NSLOT),
    )
    def layer(emb_hbm, src_hbm, dst_hbm, w_hbm, zeros_hbm, out_hbm, acc,
              *scr):
        c = lax.axis_index("c")
        s = lax.axis_index("s")
        sbuf = scr[0 * NSLOT:1 * NSLOT]
        dbuf = scr[1 * NSLOT:2 * NSLOT]
        wbuf = scr[2 * NSLOT:3 * NSLOT]
        dstl = scr[3 * NSLOT:4 * NSLOT]
        rows = scr[4 * NSLOT:5 * NSLOT]
        esem = scr[5 * NSLOT:6 * NSLOT]
        gsem = scr[6 * NSLOT:7 * NSLOT]
        ssem = scr[7 * NSLOT:8 * NSLOT]

        node_base = c * HALF
        tile_edge_base = s * EDGES_PER_TILE

        zr = s * ROWS_PER_TILE
        pltpu.sync_copy(zeros_hbm.at[pl.ds(zr, ROWS_PER_TILE)],
                        acc.at[pl.ds(zr, ROWS_PER_TILE)])
        plsc.subcore_barrier()

        def start_edges(b, ic):
            eb = tile_edge_base + ic * CHUNK
            pltpu.async_copy(src_hbm.at[pl.ds(eb, CHUNK)], sbuf[b], esem[b])
            pltpu.async_copy(dst_hbm.at[pl.ds(eb, CHUNK)], dbuf[b], esem[b])
            pltpu.async_copy(w_hbm.at[pl.ds(eb, CHUNK)], wbuf[b], esem[b])

        def wait_edges(b, ic):
            eb = tile_edge_base + ic * CHUNK
            pltpu.make_async_copy(src_hbm.at[pl.ds(eb, CHUNK)], sbuf[b],
                                  esem[b]).wait()
            pltpu.make_async_copy(dst_hbm.at[pl.ds(eb, CHUNK)], dbuf[b],
                                  esem[b]).wait()
            pltpu.make_async_copy(w_hbm.at[pl.ds(eb, CHUNK)], wbuf[b],
                                  esem[b]).wait()

        def localize(b):
            for g in range(CHUNK // 16):
                sl = pl.ds(g * 16, 16)
                dl = dbuf[b][sl] - node_base
                ok = (dl >= 0) & (dl < HALF)
                dstl[b][sl] = jnp.where(ok, dl, HALF)

        def scale(b):
            for g in range(CHUNK // 16):
                w16 = wbuf[b][pl.ds(g * 16, 16)]
                for e in range(16):
                    wspl = lax.gather(
                        w16,
                        jnp.full((16, 1), e, jnp.int32),
                        lax.GatherDimensionNumbers(
                            offset_dims=(), collapsed_slice_dims=(0,),
                            start_index_map=(0,)),
                        slice_sizes=(1,),
                        mode=lax.GatherScatterMode.PROMISE_IN_BOUNDS)
                    r = g * 16 + e
                    for cc in range(4):
                        sl = pl.ds(cc * 16, 16)
                        rows[b][r, sl] = rows[b][r, sl] * wspl

        def wait_scatter(q):
            pltpu.make_async_copy(rows[q], acc.at[dstl[q]], ssem[q]).wait()

        for q in (0, 1):
            start_edges(q, q)
            wait_edges(q, q)
            pltpu.async_copy(emb_hbm.at[sbuf[q]], rows[q], gsem[q])
        start_edges(2, 2)

        def body(i, carry):
            for q in range(NSLOT):
                n = NSLOT * i + q
                q2 = (q + 2) % NSLOT

                pltpu.make_async_copy(emb_hbm.at[sbuf[q]], rows[q],
                                      gsem[q]).wait()
                localize(q)
                scale(q)
                pltpu.async_copy(rows[q], acc.at[dstl[q]], ssem[q], add=True)

                @pl.when((n >= 1) & (n + 2 < NCHUNK))
                def _drain():
                    wait_scatter(q2)

                @pl.when(n + 2 < NCHUNK)
                def _refill():
                    wait_edges(q2, n + 2)
                    pltpu.async_copy(emb_hbm.at[sbuf[q2]], rows[q2], gsem[q2])

                @pl.when(n + 3 < NCHUNK)
                def _start():
                    start_edges(q, n + 3)

            return carry

        lax.fori_loop(0, NCHUNK // NSLOT, body, 0)

        for q in range(NSLOT):
            wait_scatter(q)

        plsc.subcore_barrier()

        fb = s * ROWS_PER_TILE
        ob = c * HALF + fb

        @pl.when(s < 15)
        def _flush_full():
            pltpu.sync_copy(acc.at[pl.ds(fb, ROWS_PER_TILE)],
                            out_hbm.at[pl.ds(ob, ROWS_PER_TILE)])

        @pl.when(s == 15)
        def _flush_tail():
            pltpu.sync_copy(acc.at[pl.ds(fb, HALF - 15 * ROWS_PER_TILE)],
                            out_hbm.at[pl.ds(ob, HALF - 15 * ROWS_PER_TILE)])

    return layer(emb, src, dst, w, zeros_acc)


def _mean4(e0, e1, e2, e3):
    a0 = e0.reshape(NUM_NODES // 2, 128)
    a1 = e1.reshape(NUM_NODES // 2, 128)
    a2 = e2.reshape(NUM_NODES // 2, 128)
    a3 = e3.reshape(NUM_NODES // 2, 128)

    def body(r0, r1, r2, r3, o):
        o[...] = (r0[...] + r1[...] + r2[...] + r3[...]) * 0.25

    spec = pl.BlockSpec((1000, 128), lambda i: (i, 0))
    out = pl.pallas_call(
        body,
        grid=(NUM_NODES // 2 // 1000,),
        in_specs=[spec, spec, spec, spec],
        out_specs=spec,
        out_shape=jax.ShapeDtypeStruct((NUM_NODES // 2, 128), jnp.float32),
    )(a0, a1, a2, a3)
    return out.reshape(NUM_NODES, EMBED_DIM)


def kernel(user_emb, item_emb, edge_weight, edge_index):
    e0 = jnp.concatenate([user_emb, item_emb], axis=0)

    pad = E_PAD - NUM_EDGES
    src = jnp.concatenate([edge_index[0], jnp.zeros((pad,), jnp.int32)])
    dst = jnp.concatenate([edge_index[1], jnp.zeros((pad,), jnp.int32)])
    w = jnp.concatenate([edge_weight, jnp.zeros((pad,), jnp.float32)])

    zeros_acc = jnp.zeros((ACC_ROWS, EMBED_DIM), jnp.float32)

    e1 = _propagate_layer(e0, src, dst, w, zeros_acc)
    e2 = _propagate_layer(e1, src, dst, w, zeros_acc)
    e3 = _propagate_layer(e2, src, dst, w, zeros_acc)

    final = _mean4(e0, e1, e2, e3)
    return (final[:NUM_USERS], final[NUM_USERS:])
